# Initial kernel scaffold; baseline (speedup 1.0000x reference)
import jax
import jax.numpy as jnp
from jax.experimental import pallas as pl


def kernel(x, edge_neighbors):
    raise NotImplementedError("write your pallas kernel here")


# speedup vs baseline: 4.3139x; 4.3139x over previous
import functools

import jax
import jax.numpy as jnp
from jax import lax
from jax.experimental import pallas as pl
from jax.experimental.pallas import tpu as pltpu
from jax.experimental.pallas import tpu_sc as plsc

N_NODES = 10000
N_EDGES = 320000
D_FEAT = 128

NUM_CORES = 2
NUM_TILES = 16
CHUNK = 128
D_HALF = D_FEAT // NUM_CORES

_N_CHUNKS = -(-N_EDGES // CHUNK)
CHUNKS_PER_TILE = -(-_N_CHUNKS // NUM_TILES)
E_PAD = CHUNKS_PER_TILE * NUM_TILES * CHUNK
ZERO_ROWS = 632
ROWS_PAD = ZERO_ROWS * NUM_TILES
OUT_ROWS_LAST = N_NODES - ZERO_ROWS * (NUM_TILES - 1)


def _sc_kernel(x_lo_hbm, x_hi_hbm, src_hbm, dst_hbm, z_hbm, out_hbm,
               acc, src_v, dst_v, rows_v, sem):
    c = lax.axis_index("c")
    s = lax.axis_index("s")

    pltpu.sync_copy(z_hbm.at[pl.ds(s * ZERO_ROWS, ZERO_ROWS)],
                    acc.at[pl.ds(s * ZERO_ROWS, ZERO_ROWS)])
    plsc.subcore_barrier()

    def chunk_body(j, carry):
        e0 = (s * CHUNKS_PER_TILE + j) * CHUNK
        pltpu.sync_copy(src_hbm.at[pl.ds(e0, CHUNK)], src_v)
        pltpu.sync_copy(dst_hbm.at[pl.ds(e0, CHUNK)], dst_v)

        @pl.when(c == 0)
        def _():
            pltpu.async_copy(x_lo_hbm.at[src_v], rows_v, sem).wait()

        @pl.when(c == 1)
        def _():
            pltpu.async_copy(x_hi_hbm.at[src_v], rows_v, sem).wait()

        pltpu.sync_copy(rows_v, acc.at[dst_v], add=True)
        return carry

    lax.fori_loop(0, CHUNKS_PER_TILE, chunk_body, 0)
    plsc.subcore_barrier()

    @pl.when(s < NUM_TILES - 1)
    def _():
        pltpu.sync_copy(acc.at[pl.ds(s * ZERO_ROWS, ZERO_ROWS)],
                        out_hbm.at[c].at[pl.ds(s * ZERO_ROWS, ZERO_ROWS)])

    @pl.when(s == NUM_TILES - 1)
    def _():
        base = (NUM_TILES - 1) * ZERO_ROWS
        pltpu.sync_copy(acc.at[pl.ds(base, OUT_ROWS_LAST)],
                        out_hbm.at[c].at[pl.ds(base, OUT_ROWS_LAST)])


@jax.jit
def kernel(x, edge_neighbors):
    en = edge_neighbors.astype(jnp.int32)
    pad = E_PAD - N_EDGES
    src = jnp.concatenate([en[1], jnp.zeros((pad,), jnp.int32)])
    dst = jnp.concatenate([en[0], jnp.full((pad,), N_NODES, jnp.int32)])
    x_lo = x[:, :D_HALF]
    x_hi = x[:, D_HALF:]
    zeros = jnp.zeros((ROWS_PAD, D_HALF), jnp.float32)

    mesh = plsc.VectorSubcoreMesh(core_axis_name="c", subcore_axis_name="s")
    run = functools.partial(
        pl.kernel,
        mesh=mesh,
        compiler_params=pltpu.CompilerParams(use_tc_tiling_on_sc=False),
        out_type=jax.ShapeDtypeStruct((NUM_CORES, N_NODES, D_HALF), jnp.float32),
        scratch_types=[
            pltpu.VMEM_SHARED((ROWS_PAD, D_HALF), jnp.float32),
            pltpu.VMEM((CHUNK,), jnp.int32),
            pltpu.VMEM((CHUNK,), jnp.int32),
            pltpu.VMEM((CHUNK, D_HALF), jnp.float32),
            pltpu.SemaphoreType.DMA,
        ],
    )(_sc_kernel)
    out3 = run(x_lo, x_hi, src, dst, zeros)
    return jnp.concatenate([out3[0], out3[1]], axis=1)


# speedup vs baseline: 6.0843x; 1.4104x over previous
import functools

import jax
import jax.numpy as jnp
from jax import lax
from jax.experimental import pallas as pl
from jax.experimental.pallas import tpu as pltpu
from jax.experimental.pallas import tpu_sc as plsc

N_NODES = 10000
N_EDGES = 320000
D_FEAT = 128

NUM_CORES = 2
NUM_TILES = 16
CHUNK = 128
D_HALF = D_FEAT // NUM_CORES

CHUNKS_PER_TILE = 158
E_PAD = CHUNKS_PER_TILE * NUM_TILES * CHUNK
ZERO_ROWS = 632
ROWS_PAD = ZERO_ROWS * NUM_TILES
OUT_ROWS_LAST = N_NODES - ZERO_ROWS * (NUM_TILES - 1)


def _sc_kernel(x_lo_hbm, x_hi_hbm, src_hbm, dst_hbm, z_hbm, out_hbm,
               acc, src_v, dst_v, rows0, rows1, g0, g1):
    c = lax.axis_index("c")
    s = lax.axis_index("s")
    rows = (rows0, rows1)
    gsem = (g0, g1)

    pltpu.sync_copy(z_hbm.at[pl.ds(s * ZERO_ROWS, ZERO_ROWS)],
                    acc.at[pl.ds(s * ZERO_ROWS, ZERO_ROWS)])

    pltpu.sync_copy(src_hbm.at[s], src_v)
    pltpu.sync_copy(dst_hbm.at[s], dst_v)
    plsc.subcore_barrier()

    def gather(j, b):
        @pl.when(c == 0)
        def _():
            pltpu.async_copy(x_lo_hbm.at[src_v.at[j]], rows[b], gsem[b])

        @pl.when(c == 1)
        def _():
            pltpu.async_copy(x_hi_hbm.at[src_v.at[j]], rows[b], gsem[b])

    def gwait(b):
        pltpu.make_async_copy(x_lo_hbm.at[pl.ds(0, CHUNK)], rows[b],
                              gsem[b]).wait()

    gather(0, 0)

    def chunk_body(i, carry):
        for b in (0, 1):
            j = 2 * i + b
            gwait(b)

            @pl.when(j + 1 < CHUNKS_PER_TILE)
            def _():
                gather(j + 1, 1 - b)

            pltpu.sync_copy(rows[b], acc.at[dst_v.at[j]], add=True)
        return carry

    lax.fori_loop(0, CHUNKS_PER_TILE // 2, chunk_body, 0)
    plsc.subcore_barrier()

    @pl.when(s < NUM_TILES - 1)
    def _():
        pltpu.sync_copy(acc.at[pl.ds(s * ZERO_ROWS, ZERO_ROWS)],
                        out_hbm.at[c].at[pl.ds(s * ZERO_ROWS, ZERO_ROWS)])

    @pl.when(s == NUM_TILES - 1)
    def _():
        base = (NUM_TILES - 1) * ZERO_ROWS
        pltpu.sync_copy(acc.at[pl.ds(base, OUT_ROWS_LAST)],
                        out_hbm.at[c].at[pl.ds(base, OUT_ROWS_LAST)])


@jax.jit
def kernel(x, edge_neighbors):
    en = edge_neighbors.astype(jnp.int32)
    pad = E_PAD - N_EDGES
    src = jnp.concatenate([en[1], jnp.zeros((pad,), jnp.int32)])
    dst = jnp.concatenate([en[0], jnp.full((pad,), N_NODES, jnp.int32)])
    src = src.reshape(NUM_TILES, CHUNKS_PER_TILE, CHUNK)
    dst = dst.reshape(NUM_TILES, CHUNKS_PER_TILE, CHUNK)
    x_lo = x[:, :D_HALF]
    x_hi = x[:, D_HALF:]
    zeros = jnp.zeros((ROWS_PAD, D_HALF), jnp.float32)

    mesh = plsc.VectorSubcoreMesh(core_axis_name="c", subcore_axis_name="s")
    run = functools.partial(
        pl.kernel,
        mesh=mesh,
        compiler_params=pltpu.CompilerParams(use_tc_tiling_on_sc=False),
        out_type=jax.ShapeDtypeStruct((NUM_CORES, N_NODES, D_HALF), jnp.float32),
        scratch_types=[
            pltpu.VMEM_SHARED((ROWS_PAD, D_HALF), jnp.float32),
            pltpu.VMEM((CHUNKS_PER_TILE, CHUNK), jnp.int32),
            pltpu.VMEM((CHUNKS_PER_TILE, CHUNK), jnp.int32),
            pltpu.VMEM((CHUNK, D_HALF), jnp.float32),
            pltpu.VMEM((CHUNK, D_HALF), jnp.float32),
            pltpu.SemaphoreType.DMA,
            pltpu.SemaphoreType.DMA,
        ],
    )(_sc_kernel)
    out3 = run(x_lo, x_hi, src, dst, zeros)
    return jnp.concatenate([out3[0], out3[1]], axis=1)
